# baseline (speedup 1.0000x reference)
import functools

import jax
import jax.numpy as jnp
from jax import lax
from jax.experimental import pallas as pl
from jax.experimental.pallas import tpu as pltpu
from jax.experimental.pallas import tpu_sc as plsc

V = 1000000
D = 64
B = 4096
L = 50
TOTAL = B * L
NC = 2
NS = 16
NW = NC * NS

BLK = 128
NBLK = V // BLK
TAIL = V - NBLK * BLK
BLK_PER_W = NBLK // NW
NEXTRA = NBLK - BLK_PER_W * NW

S_PER_W = B // NW
NBUF = 4
NGROUP = S_PER_W // NBUF
_LOADS = [(0, range(0, 16)), (16, range(0, 16)), (32, range(0, 16)),
          (34, range(14, 16))]


def _transpose_block(src_v, dst_v, ncols):
    rows16 = [lax.iota(jnp.int32, 16) + 16 * k for k in range(4)]

    def row_body(r, carry):
        cols = jnp.full((16,), 0, jnp.int32) + r
        for k in range(4):
            vals = plsc.load_gather(src_v, [rows16[k], cols])
            dst_v[r, pl.ds(16 * k, 16)] = vals
        return carry

    lax.fori_loop(0, ncols, row_body, 0)


def _transpose_body(tt_hbm, out_hbm, srcs, dsts, tail_v, tail_t, isems, osems):
    wid = lax.axis_index("s") * NC + lax.axis_index("c")
    base = wid * BLK_PER_W

    def start_in(j, b):
        pltpu.async_copy(
            tt_hbm.at[:, pl.ds((base + j) * BLK, BLK)], srcs.at[b], isems.at[b]
        )

    def start_out(j, b):
        pltpu.async_copy(
            out_hbm.at[pl.ds((base + j) * BLK, BLK), :], dsts.at[b], osems.at[b]
        )

    def group(j, carry):
        pltpu.sync_copy(tt_hbm.at[:, pl.ds((base + j) * BLK, BLK)], srcs.at[0])
        _transpose_block(srcs.at[0], dsts.at[0], BLK)
        pltpu.sync_copy(dsts.at[0], out_hbm.at[pl.ds((base + j) * BLK, BLK), :])
        return carry

    lax.fori_loop(0, BLK_PER_W, group, 0)
    del start_in, start_out

    @pl.when(wid < NEXTRA)
    def _():
        j = NW * BLK_PER_W + wid
        pltpu.sync_copy(tt_hbm.at[:, pl.ds(j * BLK, BLK)], srcs.at[0])
        _transpose_block(srcs.at[0], dsts.at[0], BLK)
        pltpu.sync_copy(dsts.at[0], out_hbm.at[pl.ds(j * BLK, BLK), :])

    @pl.when(wid == NEXTRA)
    def _():
        for f in range(D):
            pltpu.async_copy(
                tt_hbm.at[pl.ds(f, 1), pl.ds(NBLK * BLK, TAIL)],
                tail_v.at[pl.ds(f, 1)],
                isems.at[0],
            )
        for f in range(D):
            pltpu.make_async_copy(
                tt_hbm.at[pl.ds(f, 1), pl.ds(NBLK * BLK, TAIL)],
                tail_v.at[pl.ds(f, 1)],
                isems.at[0],
            ).wait()
        _transpose_block(tail_v, tail_t, TAIL)
        for r in range(TAIL):
            pltpu.async_copy(
                tail_t.at[pl.ds(r, 1)],
                out_hbm.at[pl.ds(NBLK * BLK + r, 1), :],
                osems.at[0],
            )
        for r in range(TAIL):
            pltpu.make_async_copy(
                tail_t.at[pl.ds(r, 1)],
                out_hbm.at[pl.ds(NBLK * BLK + r, 1), :],
                osems.at[0],
            ).wait()


def _gather_body(idx_hbm, table_hbm, out_hbm, idx_v, rows_v, gsems):
    wid = lax.axis_index("s") * NC + lax.axis_index("c")
    base = wid * S_PER_W
    pltpu.sync_copy(idx_hbm.at[pl.ds(base * L, S_PER_W * L)], idx_v)

    def issue(s, b):
        w = 0
        for off, lanes in _LOADS:
            vals = idx_v[pl.ds(s * L + off, 16)]
            for j in lanes:
                pltpu.async_copy(
                    table_hbm.at[pl.ds(vals[j], 1)],
                    rows_v.at[b, pl.ds(w, 1)],
                    gsems.at[b],
                )
                w += 1

    def drain(b):
        pltpu.make_async_copy(out_hbm.at[base], rows_v.at[b], gsems.at[b]).wait()

    for b in range(NBUF):
        issue(b, b)

    def group(g, carry):
        for b in range(NBUF):
            s = g * NBUF + b
            drain(b)
            pltpu.sync_copy(rows_v.at[b], out_hbm.at[base + s])

            @pl.when(s + NBUF < S_PER_W)
            def _():
                issue(s + NBUF, b)
        return carry

    lax.fori_loop(0, NGROUP, group, 0)


@jax.jit
def _embed_lookup(idx_flat, embed_table):
    mesh = plsc.VectorSubcoreMesh(core_axis_name="c", subcore_axis_name="s")
    table_t = jnp.swapaxes(embed_table, 0, 1)

    transpose_fn = functools.partial(
        pl.kernel,
        mesh=mesh,
        out_type=jax.ShapeDtypeStruct((V, D), jnp.float32),
        scratch_types=[
            pltpu.VMEM((2, D, BLK), jnp.float32),
            pltpu.VMEM((2, BLK, D), jnp.float32),
            pltpu.VMEM((D, TAIL), jnp.float32),
            pltpu.VMEM((TAIL, D), jnp.float32),
            pltpu.SemaphoreType.DMA((2,)),
            pltpu.SemaphoreType.DMA((2,)),
        ],
        compiler_params=pltpu.CompilerParams(
            use_tc_tiling_on_sc=True, needs_layout_passes=False
        ),
    )(_transpose_body)
    table_rm = transpose_fn(table_t)

    gather_fn = functools.partial(
        pl.kernel,
        mesh=mesh,
        out_type=jax.ShapeDtypeStruct((B, L, D), jnp.float32),
        scratch_types=[
            pltpu.VMEM((S_PER_W * L,), jnp.int32),
            pltpu.VMEM((NBUF, L, D), jnp.float32),
            pltpu.SemaphoreType.DMA((NBUF,)),
        ],
        compiler_params=pltpu.CompilerParams(use_tc_tiling_on_sc=True),
    )(_gather_body)
    return gather_fn(idx_flat, table_rm)


def kernel(sentences, embed_table):
    idx_flat = sentences.reshape(TOTAL).astype(jnp.int32)
    return _embed_lookup(idx_flat, embed_table)


# speedup vs baseline: 4.0813x; 4.0813x over previous
import functools

import jax
import jax.numpy as jnp
from jax import lax
from jax.experimental import pallas as pl
from jax.experimental.pallas import tpu as pltpu
from jax.experimental.pallas import tpu_sc as plsc

D = 64
B = 4096
L = 50
TOTAL = B * L
NC = 2
NS = 16
NW = NC * NS
S_PER_W = B // NW
NBUF = 8
NGROUP = S_PER_W // NBUF

_LOADS = [(0, range(0, 16)), (16, range(0, 16)), (32, range(0, 16)),
          (34, range(14, 16))]


def _gather_body(idx_hbm, table_hbm, out_hbm, idx_v, rows_v, gsems, osems):
    wid = lax.axis_index("s") * NC + lax.axis_index("c")
    base = wid * S_PER_W
    pltpu.sync_copy(idx_hbm.at[pl.ds(base * L, S_PER_W * L)], idx_v)

    def issue(s, b):
        w = 0
        for off, lanes in _LOADS:
            vals = idx_v[pl.ds(s * L + off, 16)]
            for j in lanes:
                pltpu.async_copy(
                    table_hbm.at[pl.ds(vals[j], 1)],
                    rows_v.at[b, pl.ds(w, 1)],
                    gsems.at[b],
                )
                w += 1

    def drain(b):
        pltpu.make_async_copy(
            out_hbm.at[base], rows_v.at[b], gsems.at[b]
        ).wait()

    for b in range(NBUF):
        issue(b, b)

    def group(g, carry):
        for b in range(NBUF):
            s = g * NBUF + b
            drain(b)
            pltpu.async_copy(rows_v.at[b], out_hbm.at[base + s], osems.at[b])
        for b in range(NBUF):
            s = g * NBUF + b
            pltpu.make_async_copy(
                rows_v.at[b], out_hbm.at[base], osems.at[b]
            ).wait()

            @pl.when(s + NBUF < S_PER_W)
            def _():
                issue(s + NBUF, b)
        return carry

    lax.fori_loop(0, NGROUP, group, 0)


@jax.jit
def _embed_lookup(idx_flat, embed_table):
    mesh = plsc.VectorSubcoreMesh(core_axis_name="c", subcore_axis_name="s")
    fn = functools.partial(
        pl.kernel,
        mesh=mesh,
        out_type=jax.ShapeDtypeStruct((B, L, D), jnp.float32),
        scratch_types=[
            pltpu.VMEM((S_PER_W * L,), jnp.int32),
            pltpu.VMEM((NBUF, L, D), jnp.float32),
            pltpu.SemaphoreType.DMA((NBUF,)),
            pltpu.SemaphoreType.DMA((NBUF,)),
        ],
        compiler_params=pltpu.CompilerParams(use_tc_tiling_on_sc=True),
    )(_gather_body)
    return fn(idx_flat, embed_table)


def kernel(sentences, embed_table):
    idx_flat = sentences.reshape(TOTAL).astype(jnp.int32)
    return _embed_lookup(idx_flat, embed_table)


# speedup vs baseline: 4.1253x; 1.0108x over previous
import functools

import jax
import jax.numpy as jnp
from jax import lax
from jax.experimental import pallas as pl
from jax.experimental.pallas import tpu as pltpu
from jax.experimental.pallas import tpu_sc as plsc

D = 64
B = 4096
L = 50
TOTAL = B * L
NC = 2
NS = 16
NW = NC * NS
S_PER_W = B // NW
NBUF = 4
NGROUP = S_PER_W // NBUF

_LOADS = [(0, range(0, 16)), (16, range(0, 16)), (32, range(0, 16)),
          (34, range(14, 16))]


def _gather_body(idx_hbm, table_hbm, out_hbm, idx_v, rows_v, gsems):
    wid = lax.axis_index("s") * NC + lax.axis_index("c")
    base = wid * S_PER_W
    pltpu.sync_copy(idx_hbm.at[pl.ds(base * L, S_PER_W * L)], idx_v)

    def issue(s, b):
        w = 0
        for off, lanes in _LOADS:
            vals = idx_v[pl.ds(s * L + off, 16)]
            for j in lanes:
                pltpu.async_copy(
                    table_hbm.at[pl.ds(vals[j], 1)],
                    rows_v.at[b, pl.ds(w, 1)],
                    gsems.at[b],
                )
                w += 1

    def drain(b):
        pltpu.make_async_copy(
            out_hbm.at[base], rows_v.at[b], gsems.at[b]
        ).wait()

    for b in range(NBUF):
        issue(b, b)

    def group(g, carry):
        for b in range(NBUF):
            s = g * NBUF + b
            drain(b)
            pltpu.sync_copy(rows_v.at[b], out_hbm.at[base + s])

            @pl.when(s + NBUF < S_PER_W)
            def _():
                issue(s + NBUF, b)
        return carry

    lax.fori_loop(0, NGROUP, group, 0)


@jax.jit
def _embed_lookup(idx_flat, embed_table):
    mesh = plsc.VectorSubcoreMesh(core_axis_name="c", subcore_axis_name="s")
    fn = functools.partial(
        pl.kernel,
        mesh=mesh,
        out_type=jax.ShapeDtypeStruct((B, L, D), jnp.float32),
        scratch_types=[
            pltpu.VMEM((S_PER_W * L,), jnp.int32),
            pltpu.VMEM((NBUF, L, D), jnp.float32),
            pltpu.SemaphoreType.DMA((NBUF,)),
        ],
        compiler_params=pltpu.CompilerParams(use_tc_tiling_on_sc=True),
    )(_gather_body)
    return fn(idx_flat, embed_table)


def kernel(sentences, embed_table):
    idx_flat = sentences.reshape(TOTAL).astype(jnp.int32)
    return _embed_lookup(idx_flat, embed_table)
